# baseline (speedup 1.0000x reference)
import jax
import jax.numpy as jnp
from jax.experimental import pallas as pl

B, P, K = 32, 1024, 10
N = B * P
_HI = jax.lax.Precision.HIGHEST


def _leaky(v):
    return jnp.where(v > 0, v, 0.01 * v)


def _dot(a, b, prec=_HI):
    return jax.lax.dot_general(
        a, b, (((a.ndim - 1,), (0,)), ((), ())),
        precision=prec, preferred_element_type=jnp.float32)


def _hi_lo(m):
    hi = m.astype(jnp.bfloat16)
    lo = (m - hi.astype(jnp.float32)).astype(jnp.bfloat16)
    return hi, lo


def _dot_f32(a, b):
    ah, al = _hi_lo(a)
    bh, bl = _hi_lo(b)
    lhs = jnp.concatenate([ah, ah, al], axis=1)
    rhs = jnp.concatenate([bh, bl, bh], axis=0)
    return _dot(lhs, rhs, prec=None)


def _gather2(b_hilo, selT):
    c = b_hilo.shape[0] // 2
    g2 = _dot(b_hilo, selT, prec=None)
    return g2[:c] + g2[c:]


def _topk_accumulate(S, per_k):
    big = jnp.float32(jnp.inf)
    acc = None
    m = jnp.min(S, axis=0, keepdims=True)
    for k in range(K):
        selT = (S == m).astype(jnp.bfloat16)
        if k < K - 1:
            m = jnp.min(jnp.where(S > m, S, big), axis=0, keepdims=True)
        h = per_k(selT)
        acc = h if acc is None else acc + h
    return acc


def _scores_t(x_row, sq_col, xT):
    lhs = jnp.concatenate([x_row, sq_col], axis=1)
    rhs = jnp.concatenate([-2.0 * xT, jnp.ones((1, P), jnp.float32)],
                          axis=0)
    return _dot_f32(lhs, rhs)


def _body(xx_ref, xxT_ref, W1dT_ref, W1jT_ref, b1c_ref, W2T_ref, b2c_ref,
          W3T_ref, b3c_ref, Wc2dT_ref, Wc2jT_ref, bc2c_ref,
          Wl1a_ref, Wl1b_ref, bl1_ref, Wm1_ref, bm1_ref,
          Wm2_ref, bm2_ref, Wm3_ref, bm3_ref, out_ref):
    f32 = jnp.float32
    xx = xx_ref[...]
    xxT = xxT_ref[...]

    sq1 = jnp.sum(xx * xx, axis=1, keepdims=True)
    S = _scores_t(xx, sq1, xxT)
    A1T = _dot(W1dT_ref[...], xxT) + b1c_ref[...]
    B1T = _dot(W1jT_ref[...], xxT)
    B1h, B1l = _hi_lo(B1T)
    B1hl = jnp.concatenate([B1h, B1l], axis=0)
    W2T = W2T_ref[...]
    b2c = b2c_ref[...]
    W3T = W3T_ref[...]
    b3c = b3c_ref[...]

    def conv1_k(selT):
        g = _gather2(B1hl, selT)
        h = _leaky(A1T + g)
        h = _leaky(_dot_f32(W2T, h) + b2c)
        h = _leaky(_dot_f32(W3T, h) + b3c)
        return h

    x1T = _topk_accumulate(S, conv1_k)
    x1 = x1T.T

    sq2 = jnp.sum(x1 * x1, axis=1, keepdims=True)
    S = _scores_t(x1, sq2, x1T)
    A2T = _dot_f32(Wc2dT_ref[...], x1T) + bc2c_ref[...]
    B2T = _dot_f32(Wc2jT_ref[...], x1T)
    B2h, B2l = _hi_lo(B2T)
    B2hl = jnp.concatenate([B2h, B2l], axis=0)

    def conv2_k(selT):
        return _leaky(A2T + _gather2(B2hl, selT))

    x2T = _topk_accumulate(S, conv2_k)

    p1 = jnp.sum(x1, axis=0, keepdims=True) * (1.0 / P)
    p2 = jnp.sum(x2T.T, axis=0, keepdims=True) * (1.0 / P)
    o = _dot(p1, Wl1a_ref[...]) + _dot(p2, Wl1b_ref[...]) + bl1_ref[...]
    o = _leaky(_dot(o, Wm1_ref[...]) + bm1_ref[...])
    o = _leaky(_dot(o, Wm2_ref[...]) + bm2_ref[...])
    o = _dot(o, Wm3_ref[...]) + bm3_ref[...]
    out_ref[...] = o.reshape(1, 1, 40)


@jax.jit
def kernel(x, pos, batch, W1, b1, W2, b2, W3, b3, Wc2, bc2, Wl1, bl1,
           Wm1, bm1, Wm2, bm2, Wm3, bm3):
    del batch
    xx = jnp.concatenate([x, pos], axis=1)
    xxT = xx.T
    W1dT = (W1[:4] - W1[4:]).T
    W1jT = W1[4:].T
    Wc2dT = (Wc2[:64] - Wc2[64:]).T
    Wc2jT = Wc2[64:].T
    Wl1a = Wl1[:64]
    Wl1b = Wl1[64:]
    row = lambda v: v.reshape(1, -1)
    col = lambda v: v.reshape(-1, 1)

    full = lambda s: pl.BlockSpec(s, lambda b: tuple(0 for _ in s))
    out = pl.pallas_call(
        _body,
        grid=(B,),
        in_specs=[
            pl.BlockSpec((P, 4), lambda b: (b, 0)),
            pl.BlockSpec((4, P), lambda b: (0, b)),
            full((64, 4)), full((64, 4)), full((64, 1)),
            full((64, 64)), full((64, 1)),
            full((64, 64)), full((64, 1)),
            full((128, 64)), full((128, 64)), full((128, 1)),
            full((64, 1024)), full((128, 1024)), full((1, 1024)),
            full((1024, 512)), full((1, 512)),
            full((512, 256)), full((1, 256)),
            full((256, 40)), full((1, 40)),
        ],
        out_specs=pl.BlockSpec((1, 1, 40), lambda b: (b, 0, 0)),
        out_shape=jax.ShapeDtypeStruct((B, 1, 40), jnp.float32),
    )(xx, xxT, W1dT, W1jT, col(b1), W2.T, col(b2), W3.T, col(b3),
      Wc2dT, Wc2jT, col(bc2), Wl1a, Wl1b, row(bl1),
      Wm1, row(bm1), Wm2, row(bm2), Wm3, row(bm3))
    return out.reshape(B, 40)


# speedup vs baseline: 1.0803x; 1.0803x over previous
import jax
import jax.numpy as jnp
from jax.experimental import pallas as pl

B, P, K = 32, 1024, 10
N = B * P
_HI = jax.lax.Precision.HIGHEST


def _leaky(v):
    return jnp.where(v > 0, v, 0.01 * v)


def _dot(a, b, prec=_HI):
    return jax.lax.dot_general(
        a, b, (((a.ndim - 1,), (0,)), ((), ())),
        precision=prec, preferred_element_type=jnp.float32)


def _hi_lo(m):
    hi = m.astype(jnp.bfloat16)
    lo = (m - hi.astype(jnp.float32)).astype(jnp.bfloat16)
    return hi, lo


def _dot_f32(a, b):
    ah, al = _hi_lo(a)
    bh, bl = _hi_lo(b)
    lhs = jnp.concatenate([ah, ah, al], axis=1)
    rhs = jnp.concatenate([bh, bl, bh], axis=0)
    return _dot(lhs, rhs, prec=None)


def _gather2(b_hilo, selT):
    c = b_hilo.shape[0] // 2
    g2 = _dot(b_hilo, selT, prec=None)
    return g2[:c] + g2[c:]


def _topk_accumulate(S, per_k):
    acc = None
    for _ in range(K):
        m = jnp.min(S, axis=0, keepdims=True)
        selb = S == m
        S = jnp.where(selb, jnp.float32(jnp.inf), S)
        h = per_k(selb.astype(jnp.bfloat16))
        acc = h if acc is None else acc + h
    return acc


def _scores_t(x_row, sq_col, xT):
    lhs = jnp.concatenate([x_row, sq_col], axis=1)
    rhs = jnp.concatenate([-2.0 * xT, jnp.ones((1, P), jnp.float32)],
                          axis=0)
    return _dot_f32(lhs, rhs)


CPS = 2


def _body(xx_ref, xxT_ref, W1dT_ref, W1jT_ref, b1c_ref, W2T_ref, b2c_ref,
          W3T_ref, b3c_ref, Wc2dT_ref, Wc2jT_ref, bc2c_ref,
          Wl1a_ref, Wl1b_ref, bl1_ref, Wm1_ref, bm1_ref,
          Wm2_ref, bm2_ref, Wm3_ref, bm3_ref, out_ref):
    outs = []
    for c in range(CPS):
        o = _one_cloud(
            xx_ref[c * P:(c + 1) * P, :], xxT_ref[:, c * P:(c + 1) * P],
            W1dT_ref, W1jT_ref, b1c_ref, W2T_ref, b2c_ref,
            W3T_ref, b3c_ref, Wc2dT_ref, Wc2jT_ref, bc2c_ref,
            Wl1a_ref, Wl1b_ref, bl1_ref, Wm1_ref, bm1_ref,
            Wm2_ref, bm2_ref, Wm3_ref, bm3_ref)
        outs.append(o)
    out_ref[...] = jnp.concatenate(outs, axis=0).reshape(1, CPS, 40)


def _one_cloud(xx, xxT, W1dT_ref, W1jT_ref, b1c_ref, W2T_ref, b2c_ref,
               W3T_ref, b3c_ref, Wc2dT_ref, Wc2jT_ref, bc2c_ref,
               Wl1a_ref, Wl1b_ref, bl1_ref, Wm1_ref, bm1_ref,
               Wm2_ref, bm2_ref, Wm3_ref, bm3_ref):
    sq1 = jnp.sum(xx * xx, axis=1, keepdims=True)
    S = _scores_t(xx, sq1, xxT)
    A1T = _dot(W1dT_ref[...], xxT) + b1c_ref[...]
    B1T = _dot(W1jT_ref[...], xxT)
    B1h, B1l = _hi_lo(B1T)
    B1hl = jnp.concatenate([B1h, B1l], axis=0)
    W2T = W2T_ref[...]
    b2c = b2c_ref[...]
    W3T = W3T_ref[...]
    b3c = b3c_ref[...]

    def conv1_k(selT):
        g = _gather2(B1hl, selT)
        h = _leaky(A1T + g)
        h = _leaky(_dot_f32(W2T, h) + b2c)
        h = _leaky(_dot_f32(W3T, h) + b3c)
        return h

    x1T = _topk_accumulate(S, conv1_k)
    x1 = x1T.T

    sq2 = jnp.sum(x1 * x1, axis=1, keepdims=True)
    S = _scores_t(x1, sq2, x1T)
    A2T = _dot_f32(Wc2dT_ref[...], x1T) + bc2c_ref[...]
    B2T = _dot_f32(Wc2jT_ref[...], x1T)
    B2h, B2l = _hi_lo(B2T)
    B2hl = jnp.concatenate([B2h, B2l], axis=0)

    def conv2_k(selT):
        return _leaky(A2T + _gather2(B2hl, selT))

    x2T = _topk_accumulate(S, conv2_k)

    p1 = jnp.sum(x1, axis=0, keepdims=True) * (1.0 / P)
    p2 = jnp.sum(x2T.T, axis=0, keepdims=True) * (1.0 / P)
    o = _dot(p1, Wl1a_ref[...]) + _dot(p2, Wl1b_ref[...]) + bl1_ref[...]
    o = _leaky(_dot(o, Wm1_ref[...]) + bm1_ref[...])
    o = _leaky(_dot(o, Wm2_ref[...]) + bm2_ref[...])
    o = _dot(o, Wm3_ref[...]) + bm3_ref[...]
    return o


@jax.jit
def kernel(x, pos, batch, W1, b1, W2, b2, W3, b3, Wc2, bc2, Wl1, bl1,
           Wm1, bm1, Wm2, bm2, Wm3, bm3):
    del batch
    xx = jnp.concatenate([x, pos], axis=1)
    xxT = xx.T
    W1dT = (W1[:4] - W1[4:]).T
    W1jT = W1[4:].T
    Wc2dT = (Wc2[:64] - Wc2[64:]).T
    Wc2jT = Wc2[64:].T
    Wl1a = Wl1[:64]
    Wl1b = Wl1[64:]
    row = lambda v: v.reshape(1, -1)
    col = lambda v: v.reshape(-1, 1)

    full = lambda s: pl.BlockSpec(s, lambda b: tuple(0 for _ in s))
    out = pl.pallas_call(
        _body,
        grid=(B // CPS,),
        in_specs=[
            pl.BlockSpec((CPS * P, 4), lambda b: (b, 0)),
            pl.BlockSpec((4, CPS * P), lambda b: (0, b)),
            full((64, 4)), full((64, 4)), full((64, 1)),
            full((64, 64)), full((64, 1)),
            full((64, 64)), full((64, 1)),
            full((128, 64)), full((128, 64)), full((128, 1)),
            full((64, 1024)), full((128, 1024)), full((1, 1024)),
            full((1024, 512)), full((1, 512)),
            full((512, 256)), full((1, 256)),
            full((256, 40)), full((1, 40)),
        ],
        out_specs=pl.BlockSpec((1, CPS, 40), lambda b: (b, 0, 0)),
        out_shape=jax.ShapeDtypeStruct((B // CPS, CPS, 40), jnp.float32),
    )(xx, xxT, W1dT, W1jT, col(b1), W2.T, col(b2), W3.T, col(b3),
      Wc2dT, Wc2jT, col(bc2), Wl1a, Wl1b, row(bl1),
      Wm1, row(bm1), Wm2, row(bm2), Wm3, row(bm3))
    return out.reshape(B, 40)


# speedup vs baseline: 1.3591x; 1.2581x over previous
import jax
import jax.numpy as jnp
from jax.experimental import pallas as pl

B, P, K = 32, 1024, 10
N = B * P
_HI = jax.lax.Precision.HIGHEST


def _leaky(v):
    return jnp.where(v > 0, v, 0.01 * v)


def _dot(a, b, prec=_HI):
    return jax.lax.dot_general(
        a, b, (((a.ndim - 1,), (0,)), ((), ())),
        precision=prec, preferred_element_type=jnp.float32)


def _hi_lo(m):
    hi = m.astype(jnp.bfloat16)
    lo = (m - hi.astype(jnp.float32)).astype(jnp.bfloat16)
    return hi, lo


def _dot_f32(a, b):
    ah, al = _hi_lo(a)
    bh, bl = _hi_lo(b)
    lhs = jnp.concatenate([ah, ah, al], axis=1)
    rhs = jnp.concatenate([bh, bl, bh], axis=0)
    return _dot(lhs, rhs, prec=None)


def _gather2(b_hilo, selT):
    c = b_hilo.shape[0] // 2
    g2 = _dot(b_hilo, selT, prec=None)
    return g2[:c] + g2[c:]


def _topk_accumulate(S, self_h, per_k):
    ii = jax.lax.broadcasted_iota(jnp.int32, (P, P), 0)
    jj = jax.lax.broadcasted_iota(jnp.int32, (P, P), 1)
    S = jnp.where(ii == jj, jnp.float32(jnp.inf), S)
    acc = self_h
    for _ in range(K - 1):
        m = jnp.min(S, axis=0, keepdims=True)
        selb = S == m
        S = jnp.where(selb, jnp.float32(jnp.inf), S)
        acc = acc + per_k(selb.astype(jnp.bfloat16))
    return acc


def _scores_t(x_row, sq_col, xT):
    lhs = jnp.concatenate([x_row, sq_col], axis=1)
    rhs = jnp.concatenate([-2.0 * xT, jnp.ones((1, P), jnp.float32)],
                          axis=0)
    return _dot_f32(lhs, rhs)


CPS = 1


def _body(xx_ref, xxT_ref, W1dT_ref, W1jT_ref, b1c_ref, W2T_ref, b2c_ref,
          W3T_ref, b3c_ref, Wc2dT_ref, Wc2jT_ref, bc2c_ref,
          Wl1a_ref, Wl1b_ref, bl1_ref, Wm1_ref, bm1_ref,
          Wm2_ref, bm2_ref, Wm3_ref, bm3_ref, out_ref):
    outs = []
    for c in range(CPS):
        o = _one_cloud(
            xx_ref[c * P:(c + 1) * P, :], xxT_ref[:, c * P:(c + 1) * P],
            W1dT_ref, W1jT_ref, b1c_ref, W2T_ref, b2c_ref,
            W3T_ref, b3c_ref, Wc2dT_ref, Wc2jT_ref, bc2c_ref,
            Wl1a_ref, Wl1b_ref, bl1_ref, Wm1_ref, bm1_ref,
            Wm2_ref, bm2_ref, Wm3_ref, bm3_ref)
        outs.append(o)
    out_ref[...] = jnp.concatenate(outs, axis=0).reshape(1, CPS, 40)


def _one_cloud(xx, xxT, W1dT_ref, W1jT_ref, b1c_ref, W2T_ref, b2c_ref,
               W3T_ref, b3c_ref, Wc2dT_ref, Wc2jT_ref, bc2c_ref,
               Wl1a_ref, Wl1b_ref, bl1_ref, Wm1_ref, bm1_ref,
               Wm2_ref, bm2_ref, Wm3_ref, bm3_ref):
    sq1 = jnp.sum(xx * xx, axis=1, keepdims=True)
    S = _scores_t(xx, sq1, xxT)
    A1T = _dot(W1dT_ref[...], xxT) + b1c_ref[...]
    W1jT = W1jT_ref[...]
    B1T = _dot(W1jT, xxT)
    xh, xl = _hi_lo(xxT)
    xxhl = jnp.concatenate([xh, xl], axis=0)
    W2T = W2T_ref[...]
    b2c = b2c_ref[...]
    W3T = W3T_ref[...]
    b3c = b3c_ref[...]

    def conv1_mlp(g):
        h = _leaky(A1T + g)
        h = _leaky(_dot_f32(W2T, h) + b2c)
        h = _leaky(_dot_f32(W3T, h) + b3c)
        return h

    def conv1_k(selT):
        gxx = _gather2(xxhl, selT)
        return conv1_mlp(_dot_f32(W1jT, gxx))

    x1T = _topk_accumulate(S, conv1_mlp(B1T), conv1_k)
    x1 = x1T.T

    sq2 = jnp.sum(x1 * x1, axis=1, keepdims=True)
    S = _scores_t(x1, sq2, x1T)
    A2T = _dot_f32(Wc2dT_ref[...], x1T) + bc2c_ref[...]
    B2T = _dot_f32(Wc2jT_ref[...], x1T)
    B2h, B2l = _hi_lo(B2T)
    B2hl = jnp.concatenate([B2h, B2l], axis=0)

    def conv2_k(selT):
        return _leaky(A2T + _gather2(B2hl, selT))

    x2T = _topk_accumulate(S, _leaky(A2T + B2T), conv2_k)

    p1 = jnp.sum(x1, axis=0, keepdims=True) * (1.0 / P)
    p2 = jnp.sum(x2T.T, axis=0, keepdims=True) * (1.0 / P)
    o = _dot(p1, Wl1a_ref[...]) + _dot(p2, Wl1b_ref[...]) + bl1_ref[...]
    o = _leaky(_dot(o, Wm1_ref[...]) + bm1_ref[...])
    o = _leaky(_dot(o, Wm2_ref[...]) + bm2_ref[...])
    o = _dot(o, Wm3_ref[...]) + bm3_ref[...]
    return o


@jax.jit
def kernel(x, pos, batch, W1, b1, W2, b2, W3, b3, Wc2, bc2, Wl1, bl1,
           Wm1, bm1, Wm2, bm2, Wm3, bm3):
    del batch
    xx = jnp.concatenate([x, pos], axis=1)
    xxT = xx.T
    W1dT = (W1[:4] - W1[4:]).T
    W1jT = W1[4:].T
    Wc2dT = (Wc2[:64] - Wc2[64:]).T
    Wc2jT = Wc2[64:].T
    Wl1a = Wl1[:64]
    Wl1b = Wl1[64:]
    row = lambda v: v.reshape(1, -1)
    col = lambda v: v.reshape(-1, 1)

    full = lambda s: pl.BlockSpec(s, lambda b: tuple(0 for _ in s))
    out = pl.pallas_call(
        _body,
        grid=(B // CPS,),
        in_specs=[
            pl.BlockSpec((CPS * P, 4), lambda b: (b, 0)),
            pl.BlockSpec((4, CPS * P), lambda b: (0, b)),
            full((64, 4)), full((64, 4)), full((64, 1)),
            full((64, 64)), full((64, 1)),
            full((64, 64)), full((64, 1)),
            full((128, 64)), full((128, 64)), full((128, 1)),
            full((64, 1024)), full((128, 1024)), full((1, 1024)),
            full((1024, 512)), full((1, 512)),
            full((512, 256)), full((1, 256)),
            full((256, 40)), full((1, 40)),
        ],
        out_specs=pl.BlockSpec((1, CPS, 40), lambda b: (b, 0, 0)),
        out_shape=jax.ShapeDtypeStruct((B // CPS, CPS, 40), jnp.float32),
    )(xx, xxT, W1dT, W1jT, col(b1), W2.T, col(b2), W3.T, col(b3),
      Wc2dT, Wc2jT, col(bc2), Wl1a, Wl1b, row(bl1),
      Wm1, row(bm1), Wm2, row(bm2), Wm3, row(bm3))
    return out.reshape(B, 40)


# speedup vs baseline: 1.3612x; 1.0015x over previous
import jax
import jax.numpy as jnp
from jax.experimental import pallas as pl

B, P, K = 32, 1024, 10
N = B * P
_HI = jax.lax.Precision.HIGHEST


def _leaky(v):
    return jnp.where(v > 0, v, 0.01 * v)


def _dot(a, b, prec=_HI):
    return jax.lax.dot_general(
        a, b, (((a.ndim - 1,), (0,)), ((), ())),
        precision=prec, preferred_element_type=jnp.float32)


def _hi_lo(m):
    hi = m.astype(jnp.bfloat16)
    lo = (m - hi.astype(jnp.float32)).astype(jnp.bfloat16)
    return hi, lo


def _dot_f32(a, b):
    ah, al = _hi_lo(a)
    bh, bl = _hi_lo(b)
    lhs = jnp.concatenate([ah, ah, al], axis=1)
    rhs = jnp.concatenate([bh, bl, bh], axis=0)
    return _dot(lhs, rhs, prec=None)


def _gather2(b_hilo, selT):
    c = b_hilo.shape[0] // 2
    g2 = _dot(b_hilo, selT, prec=None)
    return g2[:c] + g2[c:]


def _topk_accumulate(S, self_h, per_k):
    ii = jax.lax.broadcasted_iota(jnp.int32, (P, P), 0)
    jj = jax.lax.broadcasted_iota(jnp.int32, (P, P), 1)
    S = jnp.where(ii == jj, jnp.float32(jnp.inf), S)
    acc = self_h
    for _ in range(K - 1):
        m = jnp.min(S, axis=0, keepdims=True)
        selb = S == m
        sel32 = selb.astype(jnp.float32)
        S = jnp.where(selb, jnp.float32(jnp.inf), S)
        acc = acc + per_k(sel32.astype(jnp.bfloat16))
    return acc


def _scores_t(x_row, sq_col, xT):
    lhs = jnp.concatenate([x_row, sq_col], axis=1)
    rhs = jnp.concatenate([-2.0 * xT, jnp.ones((1, P), jnp.float32)],
                          axis=0)
    return _dot_f32(lhs, rhs)


CPS = 1


def _body(xx_ref, xxT_ref, W1dT_ref, W1jT_ref, b1c_ref, W2T_ref, b2c_ref,
          W3T_ref, b3c_ref, Wc2dT_ref, Wc2jT_ref, bc2c_ref,
          Wl1a_ref, Wl1b_ref, bl1_ref, Wm1_ref, bm1_ref,
          Wm2_ref, bm2_ref, Wm3_ref, bm3_ref, out_ref):
    outs = []
    for c in range(CPS):
        o = _one_cloud(
            xx_ref[c * P:(c + 1) * P, :], xxT_ref[:, c * P:(c + 1) * P],
            W1dT_ref, W1jT_ref, b1c_ref, W2T_ref, b2c_ref,
            W3T_ref, b3c_ref, Wc2dT_ref, Wc2jT_ref, bc2c_ref,
            Wl1a_ref, Wl1b_ref, bl1_ref, Wm1_ref, bm1_ref,
            Wm2_ref, bm2_ref, Wm3_ref, bm3_ref)
        outs.append(o)
    out_ref[...] = jnp.concatenate(outs, axis=0).reshape(1, CPS, 40)


def _one_cloud(xx, xxT, W1dT_ref, W1jT_ref, b1c_ref, W2T_ref, b2c_ref,
               W3T_ref, b3c_ref, Wc2dT_ref, Wc2jT_ref, bc2c_ref,
               Wl1a_ref, Wl1b_ref, bl1_ref, Wm1_ref, bm1_ref,
               Wm2_ref, bm2_ref, Wm3_ref, bm3_ref):
    sq1 = jnp.sum(xx * xx, axis=1, keepdims=True)
    S = _scores_t(xx, sq1, xxT)
    A1T = _dot(W1dT_ref[...], xxT) + b1c_ref[...]
    W1jT = W1jT_ref[...]
    B1T = _dot(W1jT, xxT)
    xh, xl = _hi_lo(xxT)
    xxhl = jnp.concatenate([xh, xl], axis=0)
    W2T = W2T_ref[...]
    b2c = b2c_ref[...]
    W3T = W3T_ref[...]
    b3c = b3c_ref[...]

    def conv1_mlp(g):
        h = _leaky(A1T + g)
        h = _leaky(_dot_f32(W2T, h) + b2c)
        h = _leaky(_dot_f32(W3T, h) + b3c)
        return h

    def conv1_k(selT):
        gxx = _gather2(xxhl, selT)
        return conv1_mlp(_dot_f32(W1jT, gxx))

    x1T = _topk_accumulate(S, conv1_mlp(B1T), conv1_k)
    x1 = x1T.T

    sq2 = jnp.sum(x1 * x1, axis=1, keepdims=True)
    S = _scores_t(x1, sq2, x1T)
    A2T = _dot_f32(Wc2dT_ref[...], x1T) + bc2c_ref[...]
    B2T = _dot_f32(Wc2jT_ref[...], x1T)
    B2h, B2l = _hi_lo(B2T)
    B2hl = jnp.concatenate([B2h, B2l], axis=0)

    def conv2_k(selT):
        return _leaky(A2T + _gather2(B2hl, selT))

    x2T = _topk_accumulate(S, _leaky(A2T + B2T), conv2_k)

    p1 = jnp.sum(x1, axis=0, keepdims=True) * (1.0 / P)
    p2 = jnp.sum(x2T.T, axis=0, keepdims=True) * (1.0 / P)
    o = _dot(p1, Wl1a_ref[...]) + _dot(p2, Wl1b_ref[...]) + bl1_ref[...]
    o = _leaky(_dot(o, Wm1_ref[...]) + bm1_ref[...])
    o = _leaky(_dot(o, Wm2_ref[...]) + bm2_ref[...])
    o = _dot(o, Wm3_ref[...]) + bm3_ref[...]
    return o


@jax.jit
def kernel(x, pos, batch, W1, b1, W2, b2, W3, b3, Wc2, bc2, Wl1, bl1,
           Wm1, bm1, Wm2, bm2, Wm3, bm3):
    del batch
    xx = jnp.concatenate([x, pos], axis=1)
    xxT = xx.T
    W1dT = (W1[:4] - W1[4:]).T
    W1jT = W1[4:].T
    Wc2dT = (Wc2[:64] - Wc2[64:]).T
    Wc2jT = Wc2[64:].T
    Wl1a = Wl1[:64]
    Wl1b = Wl1[64:]
    row = lambda v: v.reshape(1, -1)
    col = lambda v: v.reshape(-1, 1)

    full = lambda s: pl.BlockSpec(s, lambda b: tuple(0 for _ in s))
    out = pl.pallas_call(
        _body,
        grid=(B // CPS,),
        in_specs=[
            pl.BlockSpec((CPS * P, 4), lambda b: (b, 0)),
            pl.BlockSpec((4, CPS * P), lambda b: (0, b)),
            full((64, 4)), full((64, 4)), full((64, 1)),
            full((64, 64)), full((64, 1)),
            full((64, 64)), full((64, 1)),
            full((128, 64)), full((128, 64)), full((128, 1)),
            full((64, 1024)), full((128, 1024)), full((1, 1024)),
            full((1024, 512)), full((1, 512)),
            full((512, 256)), full((1, 256)),
            full((256, 40)), full((1, 40)),
        ],
        out_specs=pl.BlockSpec((1, CPS, 40), lambda b: (b, 0, 0)),
        out_shape=jax.ShapeDtypeStruct((B // CPS, CPS, 40), jnp.float32),
    )(xx, xxT, W1dT, W1jT, col(b1), W2.T, col(b2), W3.T, col(b3),
      Wc2dT, Wc2jT, col(bc2), Wl1a, Wl1b, row(bl1),
      Wm1, row(bm1), Wm2, row(bm2), Wm3, row(bm3))
    return out.reshape(B, 40)
